# Initial kernel scaffold; baseline (speedup 1.0000x reference)
import jax
import jax.numpy as jnp
from jax.experimental import pallas as pl


def kernel(feat, edge_index, etype, bases0, comp0, wself0, bias0, bases1, comp1, wself1, bias1, bases2, comp2, wself2, bias2):
    raise NotImplementedError("write your pallas kernel here")


# speedup vs baseline: 2.0898x; 2.0898x over previous
import functools

import jax
import jax.numpy as jnp
from jax import lax
from jax.experimental import pallas as pl
from jax.experimental.pallas import tpu as pltpu
from jax.experimental.pallas import tpu_sc as plsc

_N = 10000
_E = 160000
_D = 256
_R = 8
_NB = 4
_H = 128

_BLK = 400
_NBLK = _N // _BLK

_CH = 80
_EPT = _E // 16
_NIT = _EPT // _CH
_ZR = 80
_RPT = 640


def _project_body(h_ref, bases_ref, comp_ref, wself_ref, bias_ref,
                  xr_ref, self_ref):
    h = h_ref[...]
    dn = (((1,), (0,)), ((), ()))
    xb = [lax.dot_general(h, bases_ref[b], dn,
                          preferred_element_type=jnp.float32)
          for b in range(_NB)]
    self_ref[...] = (lax.dot_general(h, wself_ref[...], dn,
                                     preferred_element_type=jnp.float32)
                     + bias_ref[...])
    for r in range(_R):
        xr = xb[0] * comp_ref[r, 0]
        for b in range(1, _NB):
            xr = xr + xb[b] * comp_ref[r, b]
        xr_ref[0, r] = xr[:, :_H]
        xr_ref[1, r] = xr[:, _H:]


_project = pl.pallas_call(
    _project_body,
    grid=(_NBLK,),
    in_specs=[
        pl.BlockSpec((_BLK, _D), lambda i: (i, 0)),
        pl.BlockSpec((_NB, _D, _D), lambda i: (0, 0, 0)),
        pl.BlockSpec((_R, _NB), lambda i: (0, 0)),
        pl.BlockSpec((_D, _D), lambda i: (0, 0)),
        pl.BlockSpec((1, _D), lambda i: (0, 0)),
    ],
    out_specs=[
        pl.BlockSpec((2, _R, _BLK, _H), lambda i: (0, 0, i, 0)),
        pl.BlockSpec((_BLK, _D), lambda i: (i, 0)),
    ],
    out_shape=[
        jax.ShapeDtypeStruct((2, _R, _N, _H), jnp.float32),
        jax.ShapeDtypeStruct((_N, _D), jnp.float32),
    ],
)


def _combine_body(agg_ref, self_ref, h_ref, out_ref):
    a = jnp.concatenate([agg_ref[0], agg_ref[1]], axis=1)
    out_ref[...] = jnp.maximum(a + self_ref[...], 0.0) + h_ref[...]


_combine = pl.pallas_call(
    _combine_body,
    grid=(_NBLK,),
    in_specs=[
        pl.BlockSpec((2, _BLK, _H), lambda i: (0, i, 0)),
        pl.BlockSpec((_BLK, _D), lambda i: (i, 0)),
        pl.BlockSpec((_BLK, _D), lambda i: (i, 0)),
    ],
    out_specs=pl.BlockSpec((_BLK, _D), lambda i: (i, 0)),
    out_shape=jax.ShapeDtypeStruct((_N, _D), jnp.float32),
)


_sc_mesh = plsc.VectorSubcoreMesh(core_axis_name="c", subcore_axis_name="s")


@functools.partial(
    pl.kernel,
    mesh=_sc_mesh,
    out_type=jax.ShapeDtypeStruct((2 * _N, _H), jnp.float32),
    scratch_types=[
        pltpu.VMEM((_CH,), jnp.int32),
        pltpu.VMEM((_CH,), jnp.int32),
        pltpu.VMEM((_CH, _H), jnp.float32),
        pltpu.VMEM((_ZR, _H), jnp.float32),
        pltpu.VMEM_SHARED((_N, _H), jnp.float32),
        pltpu.SemaphoreType.DMA,
    ],
)
def _sc_aggregate(xr_hbm, gidx_hbm, dst_hbm, zeros_hbm, out_hbm,
                  idx_v, dst_v, rows_v, zbuf, acc, sem):
    c = lax.axis_index("c")
    s = lax.axis_index("s")

    pltpu.sync_copy(zeros_hbm, zbuf)
    for j in range(_RPT // _ZR):
        r0 = s * _RPT + j * _ZR

        @pl.when(r0 < _N)
        def _():
            pltpu.sync_copy(zbuf, acc.at[pl.ds(r0, _ZR)])

    plsc.subcore_barrier()

    def body(i, carry):
        base = s * _EPT + i * _CH
        pltpu.sync_copy(gidx_hbm.at[pl.ds(c * _E + base, _CH)], idx_v)
        pltpu.sync_copy(dst_hbm.at[pl.ds(base, _CH)], dst_v)
        pltpu.async_copy(xr_hbm.at[idx_v], rows_v, sem).wait()
        pltpu.sync_copy(rows_v, acc.at[dst_v], add=True)
        return carry

    lax.fori_loop(0, _NIT, body, 0)
    plsc.subcore_barrier()

    for j in range(_RPT // _ZR):
        r0 = s * _RPT + j * _ZR

        @pl.when(r0 < _N)
        def _():
            pltpu.sync_copy(acc.at[pl.ds(r0, _ZR)], zbuf)
            pltpu.sync_copy(zbuf, out_hbm.at[pl.ds(c * _N + r0, _ZR)])


def kernel(feat, edge_index, etype, bases0, comp0, wself0, bias0,
           bases1, comp1, wself1, bias1, bases2, comp2, wself2, bias2):
    src = edge_index[0]
    dst = edge_index[1]
    g = etype * _N + src
    gidx = jnp.concatenate([g, g + _R * _N])
    zeros_rows = jnp.zeros((_ZR, _H), jnp.float32)

    h = feat
    for bases, comp, wself, bias in ((bases0, comp0, wself0, bias0),
                                     (bases1, comp1, wself1, bias1),
                                     (bases2, comp2, wself2, bias2)):
        xr, selfo = _project(h, bases, comp, wself, bias.reshape(1, _D))
        agg = _sc_aggregate(xr.reshape(2 * _R * _N, _H), gidx, dst,
                            zeros_rows)
        h = _combine(agg.reshape(2, _N, _H), selfo, h)
    return h
